# baseline (speedup 1.0000x reference)
import functools

import jax
import jax.numpy as jnp
from jax import lax
from jax.experimental import pallas as pl
from jax.experimental.pallas import tpu as pltpu
from jax.experimental.pallas import tpu_sc as plsc

_NUM_EMB = 512
_DIM = 32
_COMMIT = 0.25
_TBLK = 8192
_NW = 32
_CHUNK = 1024


def _argmin_block_kernel(x_ref, emb_ref, idx_ref, loss_ref):
    x = x_ref[0]
    emb = emb_ref[...]
    flat_sq = jnp.sum(x * x, axis=0, keepdims=True)
    e_sq = jnp.sum(emb * emb, axis=1, keepdims=True)
    cross2 = jnp.dot(emb + emb, x, preferred_element_type=jnp.float32)
    m8 = (flat_sq + e_sq[0:8]) - cross2[0:8]
    r8 = jnp.zeros((8, _TBLK), jnp.int32)
    for r in range(1, _NUM_EMB // 8):
        d_r = (flat_sq + e_sq[8 * r:8 * r + 8]) - cross2[8 * r:8 * r + 8]
        lt = d_r < m8
        m8 = jnp.where(lt, d_r, m8)
        r8 = jnp.where(lt, r, r8)
    sub8 = jax.lax.broadcasted_iota(jnp.int32, (8, _TBLK), 0)
    cand = r8 * 8 + sub8
    mf = jnp.min(m8, axis=0, keepdims=True)
    idx = jnp.min(jnp.where(m8 == mf, cand, _NUM_EMB), axis=0, keepdims=True)
    idx_ref[0, 0, 0] = idx[0]
    loss_ref[...] = jnp.sum(mf).reshape(1, 1, 1, 1)


def _tc_argmin(inputs, embedding):
    B, D, T = inputs.shape
    nt = T // _TBLK
    idx4, partials = pl.pallas_call(
        _argmin_block_kernel,
        grid=(B, nt),
        in_specs=[
            pl.BlockSpec((1, D, _TBLK), lambda b, t: (b, 0, t)),
            pl.BlockSpec((_NUM_EMB, D), lambda b, t: (0, 0)),
        ],
        out_specs=[
            pl.BlockSpec((1, 1, 1, _TBLK), lambda b, t: (b, t, 0, 0)),
            pl.BlockSpec((1, 1, 1, 1), lambda b, t: (b, t, 0, 0)),
        ],
        out_shape=[
            jax.ShapeDtypeStruct((B, nt, 1, _TBLK), jnp.int32),
            jax.ShapeDtypeStruct((B, nt, 1, 1), jnp.float32),
        ],
    )(inputs, embedding)
    return idx4.reshape(B, T), partials


def _make_sc_kernel(B, D, T):
    halves = _NW // B
    span = T // halves
    nchunk = span // _CHUNK
    mesh = plsc.VectorSubcoreMesh(core_axis_name="c", subcore_axis_name="s")

    @functools.partial(
        pl.kernel,
        out_type=jax.ShapeDtypeStruct((B, D, T), jnp.float32),
        mesh=mesh,
        compiler_params=pltpu.CompilerParams(needs_layout_passes=False),
        scratch_types=[
            pltpu.VMEM((_NUM_EMB * D,), jnp.float32),
            pltpu.VMEM((2, D, _CHUNK), jnp.float32),
            pltpu.VMEM((D, _CHUNK), jnp.float32),
            pltpu.VMEM((2, _CHUNK), jnp.int32),
            pltpu.SemaphoreType.DMA,
            pltpu.SemaphoreType.DMA,
        ],
    )
    def sc_kernel(x_hbm, emb_hbm, idx_hbm, q_hbm,
                  embv, xv, qv, idxv, semx0, semx1):
        wid = lax.axis_index("s") * 2 + lax.axis_index("c")
        b = wid // halves
        t0 = (wid % halves) * span
        sems = (semx0, semx1)

        def fire(ci):
            buf = ci % 2
            ts = t0 + ci * _CHUNK
            cx = pltpu.async_copy(x_hbm.at[b, :, pl.ds(ts, _CHUNK)],
                                  xv.at[buf], sems[buf])
            cidx = pltpu.async_copy(idx_hbm.at[b, pl.ds(ts, _CHUNK)],
                                    idxv.at[buf], sems[buf])
            return (cx, cidx)

        pending = fire(0)
        pltpu.sync_copy(emb_hbm, embv)
        for ci in range(nchunk):
            buf = ci % 2
            ts = t0 + ci * _CHUNK
            pending[0].wait()
            pending[1].wait()
            if ci + 1 < nchunk:
                pending = fire(ci + 1)

            def body(j, carry):
                jb = j * 16
                flat16 = idxv[buf, pl.ds(jb, 16)] * D
                for g in range(0, D, 16):
                    evs = [plsc.load_gather(embv, [flat16 + d])
                           for d in range(g, g + 16)]
                    xds = [xv[buf, d, pl.ds(jb, 16)] for d in range(g, g + 16)]
                    for k, d in enumerate(range(g, g + 16)):
                        qv[d, pl.ds(jb, 16)] = xds[k] + (evs[k] - xds[k])
                return carry

            lax.fori_loop(0, _CHUNK // 16, body, 0)
            pltpu.sync_copy(qv, q_hbm.at[b, :, pl.ds(ts, _CHUNK)])

    return sc_kernel


def kernel(inputs, embedding):
    B, D, T = inputs.shape
    nsplit = 8
    bs = B // nsplit
    emb_flat = embedding.reshape(-1)
    sc = _make_sc_kernel(bs, D, T)
    q_parts, idx_parts, loss_parts = [], [], []
    for s in range(nsplit):
        x_s = lax.slice_in_dim(inputs, s * bs, (s + 1) * bs, axis=0)
        idx_s, lp_s = _tc_argmin(x_s, embedding)
        q_parts.append(sc(x_s, emb_flat, idx_s))
        idx_parts.append(idx_s)
        loss_parts.append(jnp.sum(lp_s))
    q_st = jnp.concatenate(q_parts, axis=0)
    indices = jnp.concatenate(idx_parts, axis=0)
    mse = sum(loss_parts) / (B * D * T)
    loss_vq = mse + _COMMIT * mse
    return (q_st, loss_vq, indices)


# speedup vs baseline: 1.0352x; 1.0352x over previous
import functools

import jax
import jax.numpy as jnp
from jax import lax
from jax.experimental import pallas as pl
from jax.experimental.pallas import tpu as pltpu
from jax.experimental.pallas import tpu_sc as plsc

_NUM_EMB = 512
_DIM = 32
_COMMIT = 0.25
_TBLK = 8192
_NW = 32
_CHUNK = 1024


def _argmin_block_kernel(x_ref, emb_ref, idx_ref, loss_ref):
    x = x_ref[0]
    emb = emb_ref[...]
    flat_sq = jnp.sum(x * x, axis=0, keepdims=True)
    e_sq = jnp.sum(emb * emb, axis=1, keepdims=True)
    cross2 = jnp.dot(emb + emb, x, preferred_element_type=jnp.float32)
    m8 = (flat_sq + e_sq[0:8]) - cross2[0:8]
    r8 = jnp.zeros((8, _TBLK), jnp.int32)
    for r in range(1, _NUM_EMB // 8):
        d_r = (flat_sq + e_sq[8 * r:8 * r + 8]) - cross2[8 * r:8 * r + 8]
        lt = d_r < m8
        m8 = jnp.where(lt, d_r, m8)
        r8 = jnp.where(lt, r, r8)
    sub8 = jax.lax.broadcasted_iota(jnp.int32, (8, _TBLK), 0)
    cand = r8 * 8 + sub8
    mf = jnp.min(m8, axis=0, keepdims=True)
    idx = jnp.min(jnp.where(m8 == mf, cand, _NUM_EMB), axis=0, keepdims=True)
    idx_ref[0, 0, 0] = idx[0]
    loss_ref[...] = jnp.sum(mf).reshape(1, 1, 1, 1)


def _tc_argmin(inputs, embedding):
    B, D, T = inputs.shape
    nt = T // _TBLK
    idx4, partials = pl.pallas_call(
        _argmin_block_kernel,
        grid=(B, nt),
        in_specs=[
            pl.BlockSpec((1, D, _TBLK), lambda b, t: (b, 0, t)),
            pl.BlockSpec((_NUM_EMB, D), lambda b, t: (0, 0)),
        ],
        out_specs=[
            pl.BlockSpec((1, 1, 1, _TBLK), lambda b, t: (b, t, 0, 0)),
            pl.BlockSpec((1, 1, 1, 1), lambda b, t: (b, t, 0, 0)),
        ],
        out_shape=[
            jax.ShapeDtypeStruct((B, nt, 1, _TBLK), jnp.int32),
            jax.ShapeDtypeStruct((B, nt, 1, 1), jnp.float32),
        ],
    )(inputs, embedding)
    return idx4.reshape(B, T), partials


def _make_sc_kernel(B, D, T):
    halves = _NW // B
    span = T // halves
    nchunk = span // _CHUNK
    mesh = plsc.VectorSubcoreMesh(core_axis_name="c", subcore_axis_name="s")

    @functools.partial(
        pl.kernel,
        out_type=jax.ShapeDtypeStruct((B, D, T), jnp.float32),
        mesh=mesh,
        compiler_params=pltpu.CompilerParams(needs_layout_passes=False),
        scratch_types=[
            pltpu.VMEM((_NUM_EMB * D,), jnp.float32),
            pltpu.VMEM((2, D, _CHUNK), jnp.float32),
            pltpu.VMEM((D, _CHUNK), jnp.float32),
            pltpu.VMEM((2, _CHUNK), jnp.int32),
            pltpu.SemaphoreType.DMA,
            pltpu.SemaphoreType.DMA,
        ],
    )
    def sc_kernel(x_hbm, emb_hbm, idx_hbm, q_hbm,
                  embv, xv, qv, idxv, semx0, semx1):
        wid = lax.axis_index("s") * 2 + lax.axis_index("c")
        b = wid // halves
        t0 = (wid % halves) * span
        sems = (semx0, semx1)

        def fire(ci):
            buf = ci % 2
            ts = t0 + ci * _CHUNK
            cx = pltpu.async_copy(x_hbm.at[b, :, pl.ds(ts, _CHUNK)],
                                  xv.at[buf], sems[buf])
            cidx = pltpu.async_copy(idx_hbm.at[b, pl.ds(ts, _CHUNK)],
                                    idxv.at[buf], sems[buf])
            return (cx, cidx)

        pending = fire(0)
        pltpu.sync_copy(emb_hbm, embv)
        for ci in range(nchunk):
            buf = ci % 2
            ts = t0 + ci * _CHUNK
            pending[0].wait()
            pending[1].wait()
            if ci + 1 < nchunk:
                pending = fire(ci + 1)

            def body(j, carry):
                jb = j * 16
                flat16 = idxv[buf, pl.ds(jb, 16)] * D
                for g in range(0, D, 16):
                    evs = [plsc.load_gather(embv, [flat16 + d])
                           for d in range(g, g + 16)]
                    xds = [xv[buf, d, pl.ds(jb, 16)] for d in range(g, g + 16)]
                    for k, d in enumerate(range(g, g + 16)):
                        qv[d, pl.ds(jb, 16)] = xds[k] + (evs[k] - xds[k])
                return carry

            lax.fori_loop(0, _CHUNK // 16, body, 0)
            pltpu.sync_copy(qv, q_hbm.at[b, :, pl.ds(ts, _CHUNK)])

    return sc_kernel


def kernel(inputs, embedding):
    B, D, T = inputs.shape
    nsplit = 4
    bs = B // nsplit
    emb_flat = embedding.reshape(-1)
    sc = _make_sc_kernel(bs, D, T)
    q_parts, idx_parts, loss_parts = [], [], []
    for s in range(nsplit):
        x_s = lax.slice_in_dim(inputs, s * bs, (s + 1) * bs, axis=0)
        idx_s, lp_s = _tc_argmin(x_s, embedding)
        q_parts.append(sc(x_s, emb_flat, idx_s))
        idx_parts.append(idx_s)
        loss_parts.append(jnp.sum(lp_s))
    q_st = jnp.concatenate(q_parts, axis=0)
    indices = jnp.concatenate(idx_parts, axis=0)
    mse = sum(loss_parts) / (B * D * T)
    loss_vq = mse + _COMMIT * mse
    return (q_st, loss_vq, indices)
